# baseline (speedup 1.0000x reference)
import functools

import jax
import jax.numpy as jnp
from jax import lax
from jax.experimental import pallas as pl
from jax.experimental.pallas import tpu as pltpu
from jax.experimental.pallas import tpu_sc as plsc

B = 16384
D = 64
NC = 2
NS = 16
NW = NC * NS
PER_W = B // NW
C = 128
NCHUNK = PER_W // C
GROUPS = C // 16
NROWS = B // C


def _neg_sqrt(x):
    xi = lax.bitcast_convert_type(x, jnp.int32)
    y = lax.bitcast_convert_type(jnp.int32(0x5F3759DF) - (xi >> 1), jnp.float32)
    half = jnp.float32(0.5) * x
    for _ in range(3):
        y = y * (jnp.float32(1.5) - half * y * y)
    return -(x * y)


def _body(s_hbm, r_hbm, o_hbm, t_hbm, ent, rel, tim, out,
          sidx, ridx, oidx, tidx,
          sb0, sb1, rb0, rb1, ob0, ob1, tb0, tb1,
          outv, sem0, sem1):
    cid = lax.axis_index("c")
    sid = lax.axis_index("s")
    wid = sid * NC + cid
    row0 = wid * NCHUNK

    pltpu.sync_copy(s_hbm.at[pl.ds(row0, NCHUNK)], sidx)
    pltpu.sync_copy(r_hbm.at[pl.ds(row0, NCHUNK)], ridx)
    pltpu.sync_copy(o_hbm.at[pl.ds(row0, NCHUNK)], oidx)
    pltpu.sync_copy(t_hbm.at[pl.ds(row0, NCHUNK)], tidx)

    sbufs = (sb0, sb1)
    rbufs = (rb0, rb1)
    obufs = (ob0, ob1)
    tbufs = (tb0, tb1)
    sems = (sem0, sem1)

    def fire(chunk):
        slot = chunk % 2
        sem = sems[slot]
        return (
            pltpu.async_copy(ent.at[sidx.at[chunk]], sbufs[slot], sem),
            pltpu.async_copy(rel.at[ridx.at[chunk]], rbufs[slot], sem),
            pltpu.async_copy(ent.at[oidx.at[chunk]], obufs[slot], sem),
            pltpu.async_copy(tim.at[tidx.at[chunk]], tbufs[slot], sem),
        )

    handles = {0: fire(0)}

    for chunk in range(NCHUNK):
        slot = chunk % 2
        if chunk + 1 < NCHUNK:
            handles[chunk + 1] = fire(chunk + 1)
        for h in handles.pop(chunk):
            h.wait()

        sb, rb, ob, tb = sbufs[slot], rbufs[slot], obufs[slot], tbufs[slot]

        def group_body(g, _, sb=sb, rb=rb, ob=ob, tb=tb, chunk=chunk):
            rows = lax.iota(jnp.int32, 16) + g * 16
            acc = jnp.zeros((16,), jnp.float32)
            for j in range(D):
                col = jnp.full((16,), j, jnp.int32)
                sv = plsc.load_gather(sb, [rows, col])
                rv = plsc.load_gather(rb, [rows, col])
                tv = plsc.load_gather(tb, [rows, col])
                ov = plsc.load_gather(ob, [rows, col])
                dv = (sv + rv) + (tv - ov)
                acc = acc + dv * dv
            outv[pl.ds(chunk * C + g * 16, 16)] = _neg_sqrt(acc)
            return 0

        lax.fori_loop(0, GROUPS, group_body, 0)

    pltpu.sync_copy(outv, out.at[pl.ds(wid * PER_W, PER_W)])


_ttranse = functools.partial(
    pl.kernel,
    out_type=jax.ShapeDtypeStruct((B,), jnp.float32),
    mesh=plsc.VectorSubcoreMesh(core_axis_name="c", subcore_axis_name="s"),
    compiler_params=pltpu.CompilerParams(
        needs_layout_passes=False, use_tc_tiling_on_sc=False),
    scratch_types=[
        pltpu.VMEM((NCHUNK, C), jnp.int32),
        pltpu.VMEM((NCHUNK, C), jnp.int32),
        pltpu.VMEM((NCHUNK, C), jnp.int32),
        pltpu.VMEM((NCHUNK, C), jnp.int32),
        pltpu.VMEM((C, D), jnp.float32),
        pltpu.VMEM((C, D), jnp.float32),
        pltpu.VMEM((C, D), jnp.float32),
        pltpu.VMEM((C, D), jnp.float32),
        pltpu.VMEM((C, D), jnp.float32),
        pltpu.VMEM((C, D), jnp.float32),
        pltpu.VMEM((C, D), jnp.float32),
        pltpu.VMEM((C, D), jnp.float32),
        pltpu.VMEM((PER_W,), jnp.float32),
        pltpu.SemaphoreType.DMA,
        pltpu.SemaphoreType.DMA,
    ],
)(_body)


def kernel(input_0, input_1, input_2, input_3, entities, relations, times):
    s_idx = input_0.astype(jnp.int32).reshape(NROWS, C)
    r_idx = input_1.astype(jnp.int32).reshape(NROWS, C)
    o_idx = input_2.astype(jnp.int32).reshape(NROWS, C)
    t_idx = input_3.astype(jnp.int32).reshape(NROWS, C)
    return _ttranse(s_idx, r_idx, o_idx, t_idx, entities, relations, times)


# speedup vs baseline: 1.1226x; 1.1226x over previous
import functools

import jax
import jax.numpy as jnp
from jax import lax
from jax.experimental import pallas as pl
from jax.experimental.pallas import tpu as pltpu
from jax.experimental.pallas import tpu_sc as plsc

B = 16384
D = 64
NC = 2
NS = 16
NW = NC * NS
PER_W = B // NW
C = 128
NCHUNK = PER_W // C
GROUPS = C // 16
NROWS = B // C
ROW_UNROLL = 2


def _neg_sqrt(x):
    xi = lax.bitcast_convert_type(x, jnp.int32)
    y = lax.bitcast_convert_type(jnp.int32(0x5F3759DF) - (xi >> 1), jnp.float32)
    half = jnp.float32(0.5) * x
    for _ in range(3):
        y = y * (jnp.float32(1.5) - half * y * y)
    return -(x * y)


def _body(s_hbm, r_hbm, o_hbm, t_hbm, ent, rel, tim, out,
          sidx, ridx, oidx, tidx,
          sb0, sb1, rb0, rb1, ob0, ob1, tb0, tb1,
          psum, outv, sem0, sem1):
    cid = lax.axis_index("c")
    sid = lax.axis_index("s")
    wid = sid * NC + cid
    row0 = wid * NCHUNK

    pltpu.sync_copy(s_hbm.at[pl.ds(row0, NCHUNK)], sidx)
    pltpu.sync_copy(r_hbm.at[pl.ds(row0, NCHUNK)], ridx)
    pltpu.sync_copy(o_hbm.at[pl.ds(row0, NCHUNK)], oidx)
    pltpu.sync_copy(t_hbm.at[pl.ds(row0, NCHUNK)], tidx)

    sbufs = (sb0, sb1)
    rbufs = (rb0, rb1)
    obufs = (ob0, ob1)
    tbufs = (tb0, tb1)
    sems = (sem0, sem1)

    def fire(chunk):
        slot = chunk % 2
        sem = sems[slot]
        return (
            pltpu.async_copy(ent.at[sidx.at[chunk]], sbufs[slot], sem),
            pltpu.async_copy(rel.at[ridx.at[chunk]], rbufs[slot], sem),
            pltpu.async_copy(ent.at[oidx.at[chunk]], obufs[slot], sem),
            pltpu.async_copy(tim.at[tidx.at[chunk]], tbufs[slot], sem),
        )

    handles = {0: fire(0)}

    for chunk in range(NCHUNK):
        slot = chunk % 2
        if chunk + 1 < NCHUNK:
            handles[chunk + 1] = fire(chunk + 1)
        for h in handles.pop(chunk):
            h.wait()

        sb, rb, ob, tb = sbufs[slot], rbufs[slot], obufs[slot], tbufs[slot]

        def row_body(c, _, sb=sb, rb=rb, ob=ob, tb=tb):
            for u in range(ROW_UNROLL):
                cc = c * ROW_UNROLL + u
                acc = jnp.zeros((16,), jnp.float32)
                for k in range(D // 16):
                    sl = pl.ds(k * 16, 16)
                    dv = (sb[cc, sl] + rb[cc, sl]) + (tb[cc, sl] - ob[cc, sl])
                    acc = acc + dv * dv
                psum[pl.ds(cc * 17, 16)] = acc
            return 0

        lax.fori_loop(0, C // ROW_UNROLL, row_body, 0)

        def group_body(g, _, chunk=chunk):
            rows = lax.iota(jnp.int32, 16) + g * 16
            base = rows * 17
            tot = jnp.zeros((16,), jnp.float32)
            for j in range(16):
                tot = tot + plsc.load_gather(psum, [base + j])
            outv[pl.ds(chunk * C + g * 16, 16)] = _neg_sqrt(tot)
            return 0

        lax.fori_loop(0, GROUPS, group_body, 0)

    pltpu.sync_copy(outv, out.at[pl.ds(wid * PER_W, PER_W)])


_ttranse = functools.partial(
    pl.kernel,
    out_type=jax.ShapeDtypeStruct((B,), jnp.float32),
    mesh=plsc.VectorSubcoreMesh(core_axis_name="c", subcore_axis_name="s"),
    compiler_params=pltpu.CompilerParams(
        needs_layout_passes=False, use_tc_tiling_on_sc=False),
    scratch_types=[
        pltpu.VMEM((NCHUNK, C), jnp.int32),
        pltpu.VMEM((NCHUNK, C), jnp.int32),
        pltpu.VMEM((NCHUNK, C), jnp.int32),
        pltpu.VMEM((NCHUNK, C), jnp.int32),
        pltpu.VMEM((C, D), jnp.float32),
        pltpu.VMEM((C, D), jnp.float32),
        pltpu.VMEM((C, D), jnp.float32),
        pltpu.VMEM((C, D), jnp.float32),
        pltpu.VMEM((C, D), jnp.float32),
        pltpu.VMEM((C, D), jnp.float32),
        pltpu.VMEM((C, D), jnp.float32),
        pltpu.VMEM((C, D), jnp.float32),
        pltpu.VMEM((C * 17,), jnp.float32),
        pltpu.VMEM((PER_W,), jnp.float32),
        pltpu.SemaphoreType.DMA,
        pltpu.SemaphoreType.DMA,
    ],
)(_body)


def kernel(input_0, input_1, input_2, input_3, entities, relations, times):
    s_idx = input_0.astype(jnp.int32).reshape(NROWS, C)
    r_idx = input_1.astype(jnp.int32).reshape(NROWS, C)
    o_idx = input_2.astype(jnp.int32).reshape(NROWS, C)
    t_idx = input_3.astype(jnp.int32).reshape(NROWS, C)
    return _ttranse(s_idx, r_idx, o_idx, t_idx, entities, relations, times)
